# baseline (speedup 1.0000x reference)
import functools

import jax
import jax.numpy as jnp
import numpy as np
from jax import lax
from jax.experimental import pallas as pl
from jax.experimental.pallas import tpu as pltpu
from jax.experimental.pallas import tpu_sc as plsc

_N, _E, _D, _H = 10000, 320000, 128, 8
_DH = _D // _H
_NC, _NS = 2, 16
_NW = _NC * _NS
_EPW = _E // _NW
_C = 40
_NCHUNK = _EPW // _C
_NPAD = 10240
_XROWS = _NPAD // 8
_AROWS = _NPAD + _XROWS
_RPT = _AROWS // _NS
_PREC = lax.Precision.HIGHEST


def _qkv_body(x_ref, wq_ref, wk_ref, wv_ref, q_ref, kv_ref):
    x = x_ref[...]
    q_ref[...] = jnp.dot(x, wq_ref[...], preferred_element_type=jnp.float32,
                         precision=_PREC) * np.float32(1.0 / np.sqrt(_DH))
    kv_ref[:, :_D] = jnp.dot(x, wk_ref[...],
                             preferred_element_type=jnp.float32,
                             precision=_PREC)
    kv_ref[:, _D:] = jnp.dot(x, wv_ref[...],
                             preferred_element_type=jnp.float32,
                             precision=_PREC)


_qkv_call = pl.pallas_call(
    _qkv_body,
    out_shape=[jax.ShapeDtypeStruct((_N, _D), jnp.float32),
               jax.ShapeDtypeStruct((_N, 2 * _D), jnp.float32)],
)


_mesh = plsc.VectorSubcoreMesh(core_axis_name="c", subcore_axis_name="s")


@functools.partial(
    pl.kernel,
    out_type=jax.ShapeDtypeStruct((_NC, _AROWS, _D), jnp.float32),
    mesh=_mesh,
    scratch_types=[
        pltpu.VMEM((_C,), jnp.int32),
        pltpu.VMEM((_C,), jnp.int32),
        pltpu.VMEM((_C,), jnp.int32),
        pltpu.VMEM((_C, _D), jnp.float32),
        pltpu.VMEM((_C, 2 * _D), jnp.float32),
        pltpu.VMEM((_C, _D), jnp.float32),
        pltpu.VMEM_SHARED((_AROWS, _D), jnp.float32),
        pltpu.SemaphoreType.DMA,
        pltpu.SemaphoreType.DMA,
    ],
)
def _edge_kernel(q_hbm, kv_hbm, src_hbm, dst_hbm, xrow_hbm, out_hbm,
                 dst_v, src_v, xrow_v, q_v, kv_v, ex_v,
                 accum, sem_q, sem_kv):
    cid = lax.axis_index("c")
    sid = lax.axis_index("s")
    wid = sid * _NC + cid
    ebase = wid * _EPW

    zeros16 = jnp.zeros((16,), jnp.float32)

    def zrow(i, _):
        for j in range(_D // 16):
            q_v[i, pl.ds(j * 16, 16)] = zeros16
        return 0

    lax.fori_loop(0, _C, zrow, 0)
    for r in range(_RPT // _C):
        pltpu.sync_copy(q_v, accum.at[pl.ds(sid * _RPT + r * _C, _C)])
    plsc.subcore_barrier()

    lanes = lax.iota(jnp.int32, 16)

    def chunk_body(i, _):
        pltpu.sync_copy(dst_hbm.at[pl.ds(ebase + i * _C, _C)], dst_v)
        pltpu.sync_copy(src_hbm.at[pl.ds(ebase + i * _C, _C)], src_v)
        pltpu.sync_copy(xrow_hbm.at[pl.ds(ebase + i * _C, _C)], xrow_v)

        cq = pltpu.async_copy(q_hbm.at[dst_v], q_v, sem_q)
        ckv = pltpu.async_copy(kv_hbm.at[src_v], kv_v, sem_kv)
        cq.wait()
        ckv.wait()

        for base, jlo in ((0, 0), (16, 0), (24, 8)):
            b16 = jnp.bitwise_and(dst_v[pl.ds(base, 16)], 7)

            def edge_body(j, _, base=base, b16=b16):
                e = base + j
                ex16 = jnp.zeros((16,), jnp.float32)
                for h in range(_H):
                    q = q_v[e, pl.ds(h * _DH, _DH)]
                    k = kv_v[e, pl.ds(h * _DH, _DH)]
                    p = q * k
                    for sh in (8, 4, 2, 1):
                        p = p + p[lanes ^ sh]
                    s = jnp.minimum(jnp.maximum(p, np.float32(-5.0)),
                                    np.float32(5.0))
                    ev = jnp.exp(s)
                    v = kv_v[e, pl.ds(_D + h * _DH, _DH)]
                    q_v[e, pl.ds(h * _DH, _DH)] = v * ev
                    ex16 = jnp.where(lanes == h, ev, ex16)
                m = jnp.where(lanes == j, b16, 0)
                for sh in (8, 4, 2, 1):
                    m = m + m[lanes ^ sh]
                for blk in range(8):
                    ex_v[e, pl.ds(blk * _DH, 16)] = jnp.where(
                        m == blk, ex16, zeros16)
                return 0

            lax.fori_loop(jlo, 16, edge_body, 0)

        pltpu.sync_copy(q_v, accum.at[dst_v], add=True)
        pltpu.sync_copy(ex_v, accum.at[xrow_v], add=True)
        return 0

    lax.fori_loop(0, _NCHUNK, chunk_body, 0)

    plsc.subcore_barrier()
    pltpu.sync_copy(accum.at[pl.ds(sid * _RPT, _RPT)],
                    out_hbm.at[cid, pl.ds(sid * _RPT, _RPT)])


def _post_body(x_ref, p0_ref, p1_ref, e0_ref, e1_ref, wo_ref, bo_ref,
               w1_ref, b1_ref, w2_ref, b2_ref, g1_ref, be1_ref, g2_ref,
               be2_ref, o_ref):
    msg = p0_ref[...] + p1_ref[...]
    ssum = e0_ref[:, :_H] + e1_ref[:, :_H]
    recip = 1.0 / (ssum + 1e-16)
    sel = (lax.broadcasted_iota(jnp.int32, (_H, _D), 0)
           == lax.broadcasted_iota(jnp.int32, (_H, _D), 1) // _DH
           ).astype(jnp.float32)
    expand = jnp.dot(recip, sel, preferred_element_type=jnp.float32,
                     precision=_PREC)
    aggr = msg * expand

    x = x_ref[...]
    h = jnp.dot(aggr, wo_ref[...], preferred_element_type=jnp.float32,
                precision=_PREC) + bo_ref[...] + x
    mu = jnp.mean(h, axis=0)
    var = jnp.mean((h - mu) ** 2, axis=0)
    h = (h - mu) * lax.rsqrt(var + 1e-5) * g1_ref[...] + be1_ref[...]
    h_in2 = h
    h = jnp.dot(h, w1_ref[...], preferred_element_type=jnp.float32,
                precision=_PREC) + b1_ref[...]
    h = jnp.maximum(h, 0.0)
    h = jnp.dot(h, w2_ref[...], preferred_element_type=jnp.float32,
                precision=_PREC) + b2_ref[...]
    h = h_in2 + h
    mu = jnp.mean(h, axis=0)
    var = jnp.mean((h - mu) ** 2, axis=0)
    o_ref[...] = (h - mu) * lax.rsqrt(var + 1e-5) * g2_ref[...] + be2_ref[...]


_post_call = pl.pallas_call(
    _post_body,
    out_shape=jax.ShapeDtypeStruct((_N, _D), jnp.float32),
    compiler_params=pltpu.CompilerParams(vmem_limit_bytes=100 * 1024 * 1024),
)


def kernel(x, edge_index, Wq, Wk, Wv, Wo, bo, W1, b1, W2, b2, g1, be1, g2,
           be2):
    q, kv = _qkv_call(x, Wq, Wk, Wv)
    src = edge_index[0]
    dst = edge_index[1]
    xrow = _NPAD + lax.shift_right_logical(dst, 3)
    partials = _edge_kernel(q, kv, src, dst, xrow)
    ex0 = partials[0, _NPAD:].reshape(_NPAD, 16)[:_N]
    ex1 = partials[1, _NPAD:].reshape(_NPAD, 16)[:_N]
    return _post_call(x, partials[0, :_N], partials[1, :_N], ex0, ex1, Wo,
                      bo, W1, b1, W2, b2, g1, be1, g2, be2)


# speedup vs baseline: 1.1172x; 1.1172x over previous
import functools

import jax
import jax.numpy as jnp
import numpy as np
from jax import lax
from jax.experimental import pallas as pl
from jax.experimental.pallas import tpu as pltpu
from jax.experimental.pallas import tpu_sc as plsc

_N, _E, _D, _H = 10000, 320000, 128, 8
_DH = _D // _H
_NC, _NS = 2, 16
_NW = _NC * _NS
_EPW = _E // _NW
_C = 40
_NCHUNK = _EPW // _C
_NPAIR = _NCHUNK // 2
_NPAD = 10240
_XROWS = _NPAD // 8
_AROWS = _NPAD + _XROWS
_RPT = _AROWS // _NS
_PREC = lax.Precision.HIGHEST


def _qkv_body(x_ref, wq_ref, wk_ref, wv_ref, q_ref, k_ref, v_ref):
    x = x_ref[...]
    q_ref[...] = jnp.dot(x, wq_ref[...], preferred_element_type=jnp.float32,
                         precision=_PREC) * np.float32(1.0 / np.sqrt(_DH))
    k_ref[...] = jnp.dot(x, wk_ref[...], preferred_element_type=jnp.float32,
                         precision=_PREC)
    v_ref[...] = jnp.dot(x, wv_ref[...], preferred_element_type=jnp.float32,
                         precision=_PREC)


_qkv_call = pl.pallas_call(
    _qkv_body,
    out_shape=[jax.ShapeDtypeStruct((_N, _D), jnp.float32)] * 3,
)


_mesh = plsc.VectorSubcoreMesh(core_axis_name="c", subcore_axis_name="s")

_BUF = [
    pltpu.VMEM((_C,), jnp.int32),
    pltpu.VMEM((_C,), jnp.int32),
    pltpu.VMEM((_C,), jnp.int32),
    pltpu.VMEM((_C, _D), jnp.float32),
    pltpu.VMEM((_C, _D), jnp.float32),
    pltpu.VMEM((_C, _D), jnp.float32),
    pltpu.SemaphoreType.DMA,
    pltpu.SemaphoreType.DMA,
    pltpu.SemaphoreType.DMA,
]


@functools.partial(
    pl.kernel,
    out_type=jax.ShapeDtypeStruct((_NC, _AROWS, _D), jnp.float32),
    mesh=_mesh,
    scratch_types=_BUF + _BUF + [
        pltpu.VMEM_SHARED((_AROWS, _D), jnp.float32),
    ],
)
def _edge_kernel(q_hbm, k_hbm, v_hbm, src_hbm, dst_hbm, xrow_hbm, out_hbm,
                 dst_a, src_a, xrow_a, q_a, k_a, v_a, sqa, ska, sva,
                 dst_b, src_b, xrow_b, q_b, k_b, v_b, sqb, skb, svb,
                 accum):
    cid = lax.axis_index("c")
    sid = lax.axis_index("s")
    wid = sid * _NC + cid
    ebase = wid * _EPW

    bufa = (dst_a, src_a, xrow_a, q_a, k_a, v_a, sqa, ska, sva)
    bufb = (dst_b, src_b, xrow_b, q_b, k_b, v_b, sqb, skb, svb)

    zeros16 = jnp.zeros((16,), jnp.float32)

    def zrow(i, _):
        for j in range(_D // 16):
            q_a[i, pl.ds(j * 16, 16)] = zeros16
        return 0

    lax.fori_loop(0, _C, zrow, 0)
    for r in range(_RPT // _C):
        pltpu.sync_copy(q_a, accum.at[pl.ds(sid * _RPT + r * _C, _C)])
    plsc.subcore_barrier()

    lanes = lax.iota(jnp.int32, 16)

    def load_and_fire(i, buf):
        dst_v, src_v, xrow_v, q_v, k_v, v_v, sq, sk, sv = buf
        base = ebase + i * _C
        pltpu.sync_copy(dst_hbm.at[pl.ds(base, _C)], dst_v)
        pltpu.sync_copy(src_hbm.at[pl.ds(base, _C)], src_v)
        pltpu.sync_copy(xrow_hbm.at[pl.ds(base, _C)], xrow_v)
        pltpu.async_copy(q_hbm.at[dst_v], q_v, sq)
        pltpu.async_copy(k_hbm.at[src_v], k_v, sk)
        pltpu.async_copy(v_hbm.at[src_v], v_v, sv)

    def drain(buf):
        _, _, _, q_v, k_v, v_v, sq, sk, sv = buf
        pltpu.make_async_copy(q_hbm.at[pl.ds(0, _C)], q_v, sq).wait()
        pltpu.make_async_copy(k_hbm.at[pl.ds(0, _C)], k_v, sk).wait()
        pltpu.make_async_copy(v_hbm.at[pl.ds(0, _C)], v_v, sv).wait()

    def compute_and_scatter(buf):
        dst_v, src_v, xrow_v, q_v, k_v, v_v, _, _, _ = buf

        for base, jlo in ((0, 0), (16, 0), (24, 8)):
            b16 = jnp.bitwise_and(dst_v[pl.ds(base, 16)], 7)

            def edge_body(j, _, base=base, b16=b16):
                e = base + j
                ex16 = jnp.zeros((16,), jnp.float32)
                for h in range(_H):
                    q = q_v[e, pl.ds(h * _DH, _DH)]
                    k = k_v[e, pl.ds(h * _DH, _DH)]
                    p = q * k
                    for sh in (8, 4, 2, 1):
                        p = p + p[lanes ^ sh]
                    s = jnp.minimum(jnp.maximum(p, np.float32(-5.0)),
                                    np.float32(5.0))
                    ev = jnp.exp(s)
                    v = v_v[e, pl.ds(h * _DH, _DH)]
                    q_v[e, pl.ds(h * _DH, _DH)] = v * ev
                    ex16 = jnp.where(lanes == h, ev, ex16)
                m = jnp.where(lanes == j, b16, 0)
                for sh in (8, 4, 2, 1):
                    m = m + m[lanes ^ sh]
                for blk in range(8):
                    k_v[e, pl.ds(blk * _DH, 16)] = jnp.where(
                        m == blk, ex16, zeros16)
                return 0

            lax.fori_loop(jlo, 16, edge_body, 0)

        pltpu.sync_copy(q_v, accum.at[dst_v], add=True)
        pltpu.sync_copy(k_v, accum.at[xrow_v], add=True)

    load_and_fire(0, bufa)

    def pair_body(p, _):
        i = 2 * p
        load_and_fire(i + 1, bufb)
        drain(bufa)
        compute_and_scatter(bufa)

        @pl.when(p < _NPAIR - 1)
        def _prefetch_a():
            load_and_fire(i + 2, bufa)

        drain(bufb)
        compute_and_scatter(bufb)
        return 0

    lax.fori_loop(0, _NPAIR, pair_body, 0)

    plsc.subcore_barrier()
    pltpu.sync_copy(accum.at[pl.ds(sid * _RPT, _RPT)],
                    out_hbm.at[cid, pl.ds(sid * _RPT, _RPT)])


def _post_body(x_ref, p0_ref, p1_ref, e0_ref, e1_ref, wo_ref, bo_ref,
               w1_ref, b1_ref, w2_ref, b2_ref, g1_ref, be1_ref, g2_ref,
               be2_ref, o_ref):
    msg = p0_ref[...] + p1_ref[...]
    ssum = e0_ref[:, :_H] + e1_ref[:, :_H]
    recip = 1.0 / (ssum + 1e-16)
    sel = (lax.broadcasted_iota(jnp.int32, (_H, _D), 0)
           == lax.broadcasted_iota(jnp.int32, (_H, _D), 1) // _DH
           ).astype(jnp.float32)
    expand = jnp.dot(recip, sel, preferred_element_type=jnp.float32,
                     precision=_PREC)
    aggr = msg * expand

    x = x_ref[...]
    h = jnp.dot(aggr, wo_ref[...], preferred_element_type=jnp.float32,
                precision=_PREC) + bo_ref[...] + x
    mu = jnp.mean(h, axis=0)
    var = jnp.mean((h - mu) ** 2, axis=0)
    h = (h - mu) * lax.rsqrt(var + 1e-5) * g1_ref[...] + be1_ref[...]
    h_in2 = h
    h = jnp.dot(h, w1_ref[...], preferred_element_type=jnp.float32,
                precision=_PREC) + b1_ref[...]
    h = jnp.maximum(h, 0.0)
    h = jnp.dot(h, w2_ref[...], preferred_element_type=jnp.float32,
                precision=_PREC) + b2_ref[...]
    h = h_in2 + h
    mu = jnp.mean(h, axis=0)
    var = jnp.mean((h - mu) ** 2, axis=0)
    o_ref[...] = (h - mu) * lax.rsqrt(var + 1e-5) * g2_ref[...] + be2_ref[...]


_post_call = pl.pallas_call(
    _post_body,
    out_shape=jax.ShapeDtypeStruct((_N, _D), jnp.float32),
    compiler_params=pltpu.CompilerParams(vmem_limit_bytes=100 * 1024 * 1024),
)


def kernel(x, edge_index, Wq, Wk, Wv, Wo, bo, W1, b1, W2, b2, g1, be1, g2,
           be2):
    q, k, v = _qkv_call(x, Wq, Wk, Wv)
    src = edge_index[0]
    dst = edge_index[1]
    xrow = _NPAD + lax.shift_right_logical(dst, 3)
    partials = _edge_kernel(q, k, v, src, dst, xrow)
    ex0 = partials[0, _NPAD:].reshape(_NPAD, 16)[:_N]
    ex1 = partials[1, _NPAD:].reshape(_NPAD, 16)[:_N]
    return _post_call(x, partials[0, :_N], partials[1, :_N], ex0, ex1, Wo,
                      bo, W1, b1, W2, b2, g1, be1, g2, be2)


# speedup vs baseline: 2.1818x; 1.9530x over previous
import functools

import jax
import jax.numpy as jnp
import numpy as np
from jax import lax
from jax.experimental import pallas as pl
from jax.experimental.pallas import tpu as pltpu
from jax.experimental.pallas import tpu_sc as plsc

_N, _E, _D, _H = 10000, 320000, 128, 8
_DH = _D // _H
_NC, _NS = 2, 16
_NW = _NC * _NS
_EPW = _E // _NW
_C = 40
_NCHUNK = _EPW // _C
_NPAIR = _NCHUNK // 2
_NPAD = 10240
_XROWS = _NPAD // 8
_AROWS = _NPAD + _XROWS
_RPT = _AROWS // _NS
_PREC = lax.Precision.HIGHEST


def _qkv_body(x_ref, wq_ref, wk_ref, wv_ref, q_ref, k_ref, v_ref):
    x = x_ref[...]
    q_ref[...] = jnp.dot(x, wq_ref[...], preferred_element_type=jnp.float32,
                         precision=_PREC) * np.float32(1.0 / np.sqrt(_DH))
    k_ref[...] = jnp.dot(x, wk_ref[...], preferred_element_type=jnp.float32,
                         precision=_PREC)
    v_ref[...] = jnp.dot(x, wv_ref[...], preferred_element_type=jnp.float32,
                         precision=_PREC)


_qkv_call = pl.pallas_call(
    _qkv_body,
    out_shape=[jax.ShapeDtypeStruct((_N, _D), jnp.float32)] * 3,
)


_mesh = plsc.VectorSubcoreMesh(core_axis_name="c", subcore_axis_name="s")

_BUF = [
    pltpu.VMEM((_C,), jnp.int32),
    pltpu.VMEM((_C,), jnp.int32),
    pltpu.VMEM((_C,), jnp.int32),
    pltpu.VMEM((_C, _D), jnp.float32),
    pltpu.VMEM((_C, _D), jnp.float32),
    pltpu.VMEM((_C, _D), jnp.float32),
    pltpu.SemaphoreType.DMA,
    pltpu.SemaphoreType.DMA,
    pltpu.SemaphoreType.DMA,
]


@functools.partial(
    pl.kernel,
    out_type=jax.ShapeDtypeStruct((_NC, _AROWS, _D), jnp.float32),
    mesh=_mesh,
    scratch_types=_BUF + _BUF + [
        pltpu.VMEM_SHARED((_AROWS, _D), jnp.float32),
    ],
)
def _edge_kernel(q_hbm, k_hbm, v_hbm, src_hbm, dst_hbm, xrow_hbm, out_hbm,
                 dst_a, src_a, xrow_a, q_a, k_a, v_a, sqa, ska, sva,
                 dst_b, src_b, xrow_b, q_b, k_b, v_b, sqb, skb, svb,
                 accum):
    cid = lax.axis_index("c")
    sid = lax.axis_index("s")
    wid = sid * _NC + cid
    ebase = wid * _EPW

    bufa = (dst_a, src_a, xrow_a, q_a, k_a, v_a, sqa, ska, sva)
    bufb = (dst_b, src_b, xrow_b, q_b, k_b, v_b, sqb, skb, svb)

    zeros16 = jnp.zeros((16,), jnp.float32)

    def zrow(i, _):
        for j in range(_D // 16):
            q_a[i, pl.ds(j * 16, 16)] = zeros16
        return 0

    lax.fori_loop(0, _C, zrow, 0)
    for r in range(_RPT // _C):
        pltpu.sync_copy(q_a, accum.at[pl.ds(sid * _RPT + r * _C, _C)])
    plsc.subcore_barrier()

    lanes = lax.iota(jnp.int32, 16)
    mask0 = jnp.bitwise_and(lanes, 1) == 0
    mask1 = jnp.bitwise_and(lanes, 2) == 0
    mask2 = jnp.bitwise_and(lanes, 4) == 0
    hidx = [lanes * 0 + h for h in range(_H)]

    def load_and_fire(i, buf):
        dst_v, src_v, xrow_v, q_v, k_v, v_v, sq, sk, sv = buf
        base = ebase + i * _C
        pltpu.sync_copy(dst_hbm.at[pl.ds(base, _C)], dst_v)
        pltpu.sync_copy(src_hbm.at[pl.ds(base, _C)], src_v)
        pltpu.sync_copy(xrow_hbm.at[pl.ds(base, _C)], xrow_v)
        pltpu.async_copy(q_hbm.at[dst_v], q_v, sq)
        pltpu.async_copy(k_hbm.at[src_v], k_v, sk)
        pltpu.async_copy(v_hbm.at[src_v], v_v, sv)

    def drain(buf):
        _, _, _, q_v, k_v, v_v, sq, sk, sv = buf
        pltpu.make_async_copy(q_hbm.at[pl.ds(0, _C)], q_v, sq).wait()
        pltpu.make_async_copy(k_hbm.at[pl.ds(0, _C)], k_v, sk).wait()
        pltpu.make_async_copy(v_hbm.at[pl.ds(0, _C)], v_v, sv).wait()

    def compute_and_scatter(buf):
        dst_v, src_v, xrow_v, q_v, k_v, v_v, _, _, _ = buf

        for base, jlo in ((0, 0), (16, 0), (24, 8)):
            b16 = jnp.bitwise_and(dst_v[pl.ds(base, 16)], 7)

            def edge_body(j, _, base=base, b16=b16):
                e = base + j
                ps = [q_v[e, pl.ds(h * _DH, _DH)]
                      * k_v[e, pl.ds(h * _DH, _DH)] for h in range(_H)]
                t = [p + p[lanes ^ 1] for p in ps]
                u = [jnp.where(mask0, t[2 * g], t[2 * g + 1])
                     for g in range(4)]
                u = [a + a[lanes ^ 2] for a in u]
                w = [jnp.where(mask1, u[0], u[1]),
                     jnp.where(mask1, u[2], u[3])]
                w = [a + a[lanes ^ 4] for a in w]
                s = jnp.where(mask2, w[0], w[1])
                s = s + s[lanes ^ 8]
                s = jnp.minimum(jnp.maximum(s, np.float32(-5.0)),
                                np.float32(5.0))
                se = jnp.exp(s)
                for h in range(_H):
                    q_v[e, pl.ds(h * _DH, _DH)] = (
                        v_v[e, pl.ds(h * _DH, _DH)] * se[hidx[h]])
                m = jnp.where(lanes == j, b16, 0)
                for sh in (8, 4, 2, 1):
                    m = m + m[lanes ^ sh]
                for blk in range(8):
                    k_v[e, pl.ds(blk * _DH, 16)] = jnp.where(
                        m == blk, se, zeros16)
                return 0

            lax.fori_loop(jlo, 16, edge_body, 0)

        pltpu.sync_copy(q_v, accum.at[dst_v], add=True)
        pltpu.sync_copy(k_v, accum.at[xrow_v], add=True)

    load_and_fire(0, bufa)

    def pair_body(p, _):
        i = 2 * p
        load_and_fire(i + 1, bufb)
        drain(bufa)
        compute_and_scatter(bufa)

        @pl.when(p < _NPAIR - 1)
        def _prefetch_a():
            load_and_fire(i + 2, bufa)

        drain(bufb)
        compute_and_scatter(bufb)
        return 0

    lax.fori_loop(0, _NPAIR, pair_body, 0)

    plsc.subcore_barrier()
    pltpu.sync_copy(accum.at[pl.ds(sid * _RPT, _RPT)],
                    out_hbm.at[cid, pl.ds(sid * _RPT, _RPT)])


def _post_body(x_ref, p0_ref, p1_ref, e0_ref, e1_ref, wo_ref, bo_ref,
               w1_ref, b1_ref, w2_ref, b2_ref, g1_ref, be1_ref, g2_ref,
               be2_ref, o_ref):
    msg = p0_ref[...] + p1_ref[...]
    ssum = e0_ref[:, :_H] + e1_ref[:, :_H]
    recip = 1.0 / (ssum + 1e-16)
    sel = (lax.broadcasted_iota(jnp.int32, (_H, _D), 0)
           == lax.broadcasted_iota(jnp.int32, (_H, _D), 1) // _DH
           ).astype(jnp.float32)
    expand = jnp.dot(recip, sel, preferred_element_type=jnp.float32,
                     precision=_PREC)
    aggr = msg * expand

    x = x_ref[...]
    h = jnp.dot(aggr, wo_ref[...], preferred_element_type=jnp.float32,
                precision=_PREC) + bo_ref[...] + x
    mu = jnp.mean(h, axis=0)
    var = jnp.mean((h - mu) ** 2, axis=0)
    h = (h - mu) * lax.rsqrt(var + 1e-5) * g1_ref[...] + be1_ref[...]
    h_in2 = h
    h = jnp.dot(h, w1_ref[...], preferred_element_type=jnp.float32,
                precision=_PREC) + b1_ref[...]
    h = jnp.maximum(h, 0.0)
    h = jnp.dot(h, w2_ref[...], preferred_element_type=jnp.float32,
                precision=_PREC) + b2_ref[...]
    h = h_in2 + h
    mu = jnp.mean(h, axis=0)
    var = jnp.mean((h - mu) ** 2, axis=0)
    o_ref[...] = (h - mu) * lax.rsqrt(var + 1e-5) * g2_ref[...] + be2_ref[...]


_post_call = pl.pallas_call(
    _post_body,
    out_shape=jax.ShapeDtypeStruct((_N, _D), jnp.float32),
    compiler_params=pltpu.CompilerParams(vmem_limit_bytes=100 * 1024 * 1024),
)


def kernel(x, edge_index, Wq, Wk, Wv, Wo, bo, W1, b1, W2, b2, g1, be1, g2,
           be2):
    q, k, v = _qkv_call(x, Wq, Wk, Wv)
    src = edge_index[0]
    dst = edge_index[1]
    xrow = _NPAD + lax.shift_right_logical(dst, 3)
    partials = _edge_kernel(q, k, v, src, dst, xrow)
    ex0 = partials[0, _NPAD:].reshape(_NPAD, 16)[:_N]
    ex1 = partials[1, _NPAD:].reshape(_NPAD, 16)[:_N]
    return _post_call(x, partials[0, :_N], partials[1, :_N], ex0, ex1, Wo,
                      bo, W1, b1, W2, b2, g1, be1, g2, be2)


# speedup vs baseline: 2.2633x; 1.0374x over previous
import functools

import jax
import jax.numpy as jnp
import numpy as np
from jax import lax
from jax.experimental import pallas as pl
from jax.experimental.pallas import tpu as pltpu
from jax.experimental.pallas import tpu_sc as plsc

_N, _E, _D, _H = 10000, 320000, 128, 8
_DH = _D // _H
_NC, _NS = 2, 16
_NW = _NC * _NS
_EPW = _E // _NW
_C = 40
_NCHUNK = _EPW // _C
_NCHP = 256
_NB = 16
_NBLK = _NCHP // _NB
_PPB = _NB // 2
_TRASH = 11519
_NPAD = 10240
_XROWS = _NPAD // 8
_AROWS = _NPAD + _XROWS
_RPT = _AROWS // _NS
_PREC = lax.Precision.HIGHEST


def _qkv_body(x_ref, wq_ref, wk_ref, wv_ref, q_ref, k_ref, v_ref):
    x = x_ref[...]
    q_ref[...] = jnp.dot(x, wq_ref[...], preferred_element_type=jnp.float32,
                         precision=_PREC) * np.float32(1.0 / np.sqrt(_DH))
    k_ref[...] = jnp.dot(x, wk_ref[...], preferred_element_type=jnp.float32,
                         precision=_PREC)
    v_ref[...] = jnp.dot(x, wv_ref[...], preferred_element_type=jnp.float32,
                         precision=_PREC)


_qkv_call = pl.pallas_call(
    _qkv_body,
    out_shape=[jax.ShapeDtypeStruct((_N, _D), jnp.float32)] * 3,
)


_mesh = plsc.VectorSubcoreMesh(core_axis_name="c", subcore_axis_name="s")

_BUF = [
    pltpu.VMEM((2 * _C, _D), jnp.float32),
    pltpu.VMEM((_C, _D), jnp.float32),
    pltpu.SemaphoreType.DMA,
    pltpu.SemaphoreType.DMA,
    pltpu.SemaphoreType.DMA,
]


@functools.partial(
    pl.kernel,
    out_type=jax.ShapeDtypeStruct((_NC, _AROWS, _D), jnp.float32),
    mesh=_mesh,
    scratch_types=_BUF + _BUF + [
        pltpu.VMEM((_NB, _C), jnp.int32),
        pltpu.VMEM((_NB, _C), jnp.int32),
        pltpu.VMEM((_NB, 2 * _C), jnp.int32),
        pltpu.VMEM_SHARED((_AROWS, _D), jnp.float32),
    ],
)
def _edge_kernel(q_hbm, k_hbm, v_hbm, dstr_hbm, srcr_hbm, sidx_hbm, out_hbm,
                 qk_a, v_a, sqa, ska, sva,
                 qk_b, v_b, sqb, skb, svb,
                 dst_blk, src_blk, sidx_blk, accum):
    cid = lax.axis_index("c")
    sid = lax.axis_index("s")
    wid = sid * _NC + cid
    cbase = wid * _NCHP

    bufa = (qk_a, v_a, sqa, ska, sva)
    bufb = (qk_b, v_b, sqb, skb, svb)

    zeros16 = jnp.zeros((16,), jnp.float32)

    def zrow(i, _):
        for j in range(_D // 16):
            qk_a[i, pl.ds(j * 16, 16)] = zeros16
        return 0

    lax.fori_loop(0, 2 * _C, zrow, 0)
    for r in range(_RPT // (2 * _C)):
        pltpu.sync_copy(qk_a,
                        accum.at[pl.ds(sid * _RPT + r * 2 * _C, 2 * _C)])
    plsc.subcore_barrier()

    lanes = lax.iota(jnp.int32, 16)
    mask0 = jnp.bitwise_and(lanes, 1) == 0
    mask1 = jnp.bitwise_and(lanes, 2) == 0
    mask2 = jnp.bitwise_and(lanes, 4) == 0
    hidx = [lanes * 0 + h for h in range(_H)]

    def load_block(mb):
        row = cbase + mb * _NB
        pltpu.sync_copy(dstr_hbm.at[pl.ds(row, _NB)], dst_blk)
        pltpu.sync_copy(srcr_hbm.at[pl.ds(row, _NB)], src_blk)
        pltpu.sync_copy(sidx_hbm.at[pl.ds(row, _NB)], sidx_blk)

    def fire(r, buf):
        qk_v, v_v, sq, sk, sv = buf
        pltpu.async_copy(q_hbm.at[dst_blk.at[r]], qk_v.at[pl.ds(0, _C)], sq)
        pltpu.async_copy(k_hbm.at[src_blk.at[r]], qk_v.at[pl.ds(_C, _C)], sk)
        pltpu.async_copy(v_hbm.at[src_blk.at[r]], v_v, sv)

    def drain(buf):
        qk_v, v_v, sq, sk, sv = buf
        pltpu.make_async_copy(q_hbm.at[pl.ds(0, _C)],
                              qk_v.at[pl.ds(0, _C)], sq).wait()
        pltpu.make_async_copy(k_hbm.at[pl.ds(0, _C)],
                              qk_v.at[pl.ds(_C, _C)], sk).wait()
        pltpu.make_async_copy(v_hbm.at[pl.ds(0, _C)], v_v, sv).wait()

    def compute_and_scatter(r, buf):
        qk_v, v_v, _, _, _ = buf

        for base, jlo in ((0, 0), (16, 0), (24, 8)):
            b16 = jnp.bitwise_and(dst_blk[r, pl.ds(base, 16)], 7)

            def edge_body(j, _, base=base, b16=b16):
                e = base + j
                ps = [qk_v[e, pl.ds(h * _DH, _DH)]
                      * qk_v[_C + e, pl.ds(h * _DH, _DH)]
                      for h in range(_H)]
                t = [p + p[lanes ^ 1] for p in ps]
                u = [jnp.where(mask0, t[2 * g], t[2 * g + 1])
                     for g in range(4)]
                u = [a + a[lanes ^ 2] for a in u]
                w = [jnp.where(mask1, u[0], u[1]),
                     jnp.where(mask1, u[2], u[3])]
                w = [a + a[lanes ^ 4] for a in w]
                s = jnp.where(mask2, w[0], w[1])
                s = s + s[lanes ^ 8]
                s = jnp.minimum(jnp.maximum(s, np.float32(-5.0)),
                                np.float32(5.0))
                se = jnp.exp(s)
                for h in range(_H):
                    qk_v[e, pl.ds(h * _DH, _DH)] = (
                        v_v[e, pl.ds(h * _DH, _DH)] * se[hidx[h]])
                m = jnp.where(lanes == j, b16, 0)
                for sh in (8, 4, 2, 1):
                    m = m + m[lanes ^ sh]
                for blk in range(8):
                    qk_v[_C + e, pl.ds(blk * _DH, 16)] = jnp.where(
                        m == blk, se, zeros16)
                return 0

            lax.fori_loop(jlo, 16, edge_body, 0)

        pltpu.sync_copy(qk_v, accum.at[sidx_blk.at[r]], add=True)

    load_block(0)
    fire(0, bufa)

    def blk_body(mb, _):
        def pair_body(pp, _):
            cb = 2 * pp
            fire(cb + 1, bufb)
            drain(bufa)
            compute_and_scatter(cb, bufa)

            @pl.when(pp < _PPB - 1)
            def _prefetch_a():
                fire(cb + 2, bufa)

            drain(bufb)
            compute_and_scatter(cb + 1, bufb)
            return 0

        lax.fori_loop(0, _PPB, pair_body, 0)

        @pl.when(mb < _NBLK - 1)
        def _next_block():
            load_block(mb + 1)
            fire(0, bufa)

        return 0

    lax.fori_loop(0, _NBLK, blk_body, 0)

    plsc.subcore_barrier()
    pltpu.sync_copy(accum.at[pl.ds(sid * _RPT, _RPT)],
                    out_hbm.at[cid, pl.ds(sid * _RPT, _RPT)])


def _post_body(x_ref, p0_ref, p1_ref, e0_ref, e1_ref, wo_ref, bo_ref,
               w1_ref, b1_ref, w2_ref, b2_ref, g1_ref, be1_ref, g2_ref,
               be2_ref, o_ref):
    msg = p0_ref[...] + p1_ref[...]
    ssum = e0_ref[:, :_H] + e1_ref[:, :_H]
    recip = 1.0 / (ssum + 1e-16)
    sel = (lax.broadcasted_iota(jnp.int32, (_H, _D), 0)
           == lax.broadcasted_iota(jnp.int32, (_H, _D), 1) // _DH
           ).astype(jnp.float32)
    expand = jnp.dot(recip, sel, preferred_element_type=jnp.float32,
                     precision=_PREC)
    aggr = msg * expand

    x = x_ref[...]
    h = jnp.dot(aggr, wo_ref[...], preferred_element_type=jnp.float32,
                precision=_PREC) + bo_ref[...] + x
    mu = jnp.mean(h, axis=0)
    var = jnp.mean((h - mu) ** 2, axis=0)
    h = (h - mu) * lax.rsqrt(var + 1e-5) * g1_ref[...] + be1_ref[...]
    h_in2 = h
    h = jnp.dot(h, w1_ref[...], preferred_element_type=jnp.float32,
                precision=_PREC) + b1_ref[...]
    h = jnp.maximum(h, 0.0)
    h = jnp.dot(h, w2_ref[...], preferred_element_type=jnp.float32,
                precision=_PREC) + b2_ref[...]
    h = h_in2 + h
    mu = jnp.mean(h, axis=0)
    var = jnp.mean((h - mu) ** 2, axis=0)
    o_ref[...] = (h - mu) * lax.rsqrt(var + 1e-5) * g2_ref[...] + be2_ref[...]


_post_call = pl.pallas_call(
    _post_body,
    out_shape=jax.ShapeDtypeStruct((_N, _D), jnp.float32),
    compiler_params=pltpu.CompilerParams(vmem_limit_bytes=100 * 1024 * 1024),
)


def kernel(x, edge_index, Wq, Wk, Wv, Wo, bo, W1, b1, W2, b2, g1, be1, g2,
           be2):
    q, k, v = _qkv_call(x, Wq, Wk, Wv)
    src = edge_index[0]
    dst = edge_index[1]
    dstr = dst.reshape(_NW, _NCHUNK, _C)
    srcr = src.reshape(_NW, _NCHUNK, _C)
    xrowr = _NPAD + lax.shift_right_logical(dstr, 3)
    sidxr = jnp.concatenate([dstr, xrowr], axis=2)
    zpad = jnp.zeros((_NW, _NCHP - _NCHUNK, _C), jnp.int32)
    tpad = jnp.full((_NW, _NCHP - _NCHUNK, 2 * _C), _TRASH, jnp.int32)
    dstp = jnp.concatenate([dstr, zpad], axis=1).reshape(_NW * _NCHP, _C)
    srcp = jnp.concatenate([srcr, zpad], axis=1).reshape(_NW * _NCHP, _C)
    sidx = jnp.concatenate([sidxr, tpad], axis=1).reshape(_NW * _NCHP,
                                                          2 * _C)
    partials = _edge_kernel(q, k, v, dstp, srcp, sidx)
    ex0 = partials[0, _NPAD:].reshape(_NPAD, 16)[:_N]
    ex1 = partials[1, _NPAD:].reshape(_NPAD, 16)[:_N]
    return _post_call(x, partials[0, :_N], partials[1, :_N], ex0, ex1, Wo,
                      bo, W1, b1, W2, b2, g1, be1, g2, be2)
